# Initial kernel scaffold; baseline (speedup 1.0000x reference)
import functools

import jax
import jax.numpy as jnp
from jax import lax
from jax.experimental import pallas as pl
from jax.experimental.pallas import tpu as pltpu
from jax.experimental.pallas import tpu_sc as plsc

N_NODES = 10000
N_EDGES = 320000
NSUB = 16
CHUNK = 128
EDGES_PER_TEC = ((N_EDGES + NSUB * CHUNK - 1) // (NSUB * CHUNK)) * CHUNK
E_PAD = EDGES_PER_TEC * NSUB
N_CHUNKS = EDGES_PER_TEC // CHUNK
N_PAD = 10240
ROWS_PER_TEC = N_PAD // NSUB


def _build_sc_agg(with_deg: bool):
  f32 = jnp.float32
  out_type = [
      jax.ShapeDtypeStruct((N_PAD, 128), f32),
      jax.ShapeDtypeStruct((N_PAD, 128), f32),
  ]
  scratch = [
      pltpu.VMEM_SHARED((N_PAD, 128), f32),
      pltpu.VMEM((CHUNK,), jnp.int32),
      pltpu.VMEM((CHUNK,), jnp.int32),
      pltpu.VMEM((CHUNK, 128), f32),
      pltpu.SemaphoreType.DMA,
  ]
  if with_deg:
    out_type.append(jax.ShapeDtypeStruct((N_PAD, 1), f32))
    scratch = scratch[:1] + [pltpu.VMEM_SHARED((N_PAD, 1), f32),
                             pltpu.VMEM((CHUNK, 1), f32)] + scratch[1:]

  mesh = plsc.VectorSubcoreMesh(core_axis_name="c", subcore_axis_name="s")

  def body(*refs):
    if with_deg:
      (xa, xb, src, dst, zeros, zeros1, ones,
       suma, sumb, deg,
       acc, deg_sh, ones_v, sidx, didx, rows, sem) = refs
    else:
      (xa, xb, src, dst, zeros,
       suma, sumb,
       acc, sidx, didx, rows, sem) = refs
    c = lax.axis_index("c")
    s = lax.axis_index("s")
    r0 = s * ROWS_PER_TEC
    pltpu.sync_copy(zeros.at[pl.ds(r0, ROWS_PER_TEC)],
                    acc.at[pl.ds(r0, ROWS_PER_TEC)])
    if with_deg:
      @pl.when(c == 0)
      def _():
        pltpu.sync_copy(zeros1.at[pl.ds(r0, ROWS_PER_TEC)],
                        deg_sh.at[pl.ds(r0, ROWS_PER_TEC)])
        pltpu.sync_copy(ones, ones_v)
    plsc.subcore_barrier()

    def edge_loop(table, do_deg):
      def step(i, _):
        base = s * EDGES_PER_TEC + i * CHUNK
        pltpu.sync_copy(src.at[pl.ds(base, CHUNK)], sidx)
        pltpu.sync_copy(dst.at[pl.ds(base, CHUNK)], didx)
        pltpu.async_copy(table.at[sidx], rows, sem).wait()
        pltpu.sync_copy(rows, acc.at[didx], add=True)
        if do_deg:
          pltpu.sync_copy(ones_v, deg_sh.at[didx], add=True)
        return 0
      lax.fori_loop(0, N_CHUNKS, step, 0)

    @pl.when(c == 0)
    def _():
      edge_loop(xa, with_deg)

    @pl.when(c == 1)
    def _():
      edge_loop(xb, False)

    plsc.subcore_barrier()
    sl = pl.ds(r0, ROWS_PER_TEC)

    @pl.when(c == 0)
    def _():
      pltpu.sync_copy(acc.at[sl], suma.at[sl])
      if with_deg:
        pltpu.sync_copy(deg_sh.at[sl], deg.at[sl])

    @pl.when(c == 1)
    def _():
      pltpu.sync_copy(acc.at[sl], sumb.at[sl])

  return pl.kernel(body, out_type=out_type, mesh=mesh, scratch_types=scratch)


def _build_tc_layer(out_width: int, relu: bool, split_out: bool):
  R = 1000

  def body(suma, sumb, deg, xa, xb, wl, wr, b, *outs):
    s = jnp.concatenate([suma[...], sumb[...]], axis=1)
    x = jnp.concatenate([xa[...], xb[...]], axis=1)
    rdeg = 1.0 / jnp.maximum(deg[...], 1.0)
    z = jnp.dot(s * rdeg, wl[...], preferred_element_type=jnp.float32,
                precision=lax.Precision.HIGHEST)
    z = z + jnp.dot(x, wr[...], preferred_element_type=jnp.float32,
                    precision=lax.Precision.HIGHEST)
    z = z + b[...]
    if relu:
      z = jnp.maximum(z, 0.0)
    if split_out:
      outs[0][...] = z[:, :128]
      outs[1][...] = z[:, 128:]
    else:
      outs[0][...] = z

  if split_out:
    out_shape = [jax.ShapeDtypeStruct((N_NODES, 128), jnp.float32),
                 jax.ShapeDtypeStruct((N_NODES, 128), jnp.float32)]
    out_specs = [pl.BlockSpec((R, 128), lambda i: (i, 0)),
                 pl.BlockSpec((R, 128), lambda i: (i, 0))]
  else:
    out_shape = [jax.ShapeDtypeStruct((N_NODES, out_width), jnp.float32)]
    out_specs = [pl.BlockSpec((R, out_width), lambda i: (i, 0))]

  grid = (N_NODES // R,)
  in_specs = [
      pl.BlockSpec((R, 128), lambda i: (i, 0)),
      pl.BlockSpec((R, 128), lambda i: (i, 0)),
      pl.BlockSpec((R, 1), lambda i: (i, 0)),
      pl.BlockSpec((R, 128), lambda i: (i, 0)),
      pl.BlockSpec((R, 128), lambda i: (i, 0)),
      pl.BlockSpec((256, out_width), lambda i: (0, 0)),
      pl.BlockSpec((256, out_width), lambda i: (0, 0)),
      pl.BlockSpec((1, out_width), lambda i: (0, 0)),
  ]
  return pl.pallas_call(body, grid=grid, in_specs=in_specs,
                        out_specs=out_specs, out_shape=out_shape)


_sc_agg_deg = _build_sc_agg(with_deg=True)
_sc_agg = _build_sc_agg(with_deg=False)
_tc_mid = _build_tc_layer(256, relu=True, split_out=True)
_tc_final = _build_tc_layer(128, relu=False, split_out=False)


@jax.jit
def kernel(x0, x1, edge_index, Wl0, Wr0, b0, Wl1, Wr1, b1,
           Wl2, Wr2, b2, Wl3, Wr3, b3):
  f32 = jnp.float32
  pad_e = E_PAD - N_EDGES
  src = jnp.concatenate([edge_index[0], jnp.zeros((pad_e,), jnp.int32)])
  dst = jnp.concatenate([edge_index[1],
                         jnp.full((pad_e,), N_NODES, jnp.int32)])
  zeros = jnp.zeros((N_PAD, 128), f32)
  zeros1 = jnp.zeros((N_PAD, 1), f32)
  ones = jnp.ones((CHUNK, 1), f32)

  z128 = jnp.zeros((128, 128), f32)
  wl01 = jnp.block([[Wl0, z128], [z128, Wl1]])
  wr01 = jnp.block([[Wr0, z128], [z128, Wr1]])
  b01 = jnp.concatenate([b0, b1])[None, :]

  suma, sumb, deg = _sc_agg_deg(x0, x1, src, dst, zeros, zeros1, ones)
  degN = deg[:N_NODES]
  ha, hb = _tc_mid(suma[:N_NODES], sumb[:N_NODES], degN, x0, x1,
                   wl01, wr01, b01)
  suma, sumb = _sc_agg(ha, hb, src, dst, zeros)
  xa, xb = _tc_mid(suma[:N_NODES], sumb[:N_NODES], degN, xa=ha, xb=hb,
                   wl=Wl2, wr=Wr2, b=b2[None, :]) if False else _tc_mid(
      suma[:N_NODES], sumb[:N_NODES], degN, ha, hb, Wl2, Wr2, b2[None, :])
  suma, sumb = _sc_agg(xa, xb, src, dst, zeros)
  (out,) = _tc_final(suma[:N_NODES], sumb[:N_NODES], degN, xa, xb,
                     Wl3, Wr3, b3[None, :])
  return out


# speedup vs baseline: 3.8158x; 3.8158x over previous
import functools

import jax
import jax.numpy as jnp
from jax import lax
from jax.experimental import pallas as pl
from jax.experimental.pallas import tpu as pltpu
from jax.experimental.pallas import tpu_sc as plsc

N_NODES = 10000
N_EDGES = 320000
NSUB = 16
NTILE = 32
CHUNK = 128
E_PAD = ((N_EDGES + NTILE * CHUNK - 1) // (NTILE * CHUNK)) * (NTILE * CHUNK)
EDGES_PER_TEC = E_PAD // NSUB
N_CHUNKS = EDGES_PER_TEC // CHUNK
DEG_EDGES_PER_TEC = E_PAD // NTILE
DEG_CHUNKS = DEG_EDGES_PER_TEC // CHUNK
N_PAD = 10240
ROWS_PER_TEC = N_PAD // NSUB


def _mesh():
  return plsc.VectorSubcoreMesh(core_axis_name="c", subcore_axis_name="s",
                                num_cores=2, num_subcores=NSUB)


@functools.cache
def _build_sc_agg():
  f32 = jnp.float32
  out_type = [
      jax.ShapeDtypeStruct((N_PAD, 128), f32),
      jax.ShapeDtypeStruct((N_PAD, 128), f32),
  ]
  scratch = [
      pltpu.VMEM_SHARED((N_PAD, 128), f32),
      pltpu.VMEM((CHUNK,), jnp.int32),
      pltpu.VMEM((CHUNK,), jnp.int32),
      pltpu.VMEM((CHUNK, 128), f32),
      pltpu.SemaphoreType.DMA,
  ]

  def body(xa, xb, src, dst, zeros, suma, sumb, acc, sidx, didx, rows, sem):
    c = lax.axis_index("c")
    s = lax.axis_index("s")
    r0 = s * ROWS_PER_TEC
    sl = pl.ds(r0, ROWS_PER_TEC)
    pltpu.sync_copy(zeros.at[sl], acc.at[sl])
    plsc.subcore_barrier()

    def edge_loop(table):
      def step(i, _):
        base = s * EDGES_PER_TEC + i * CHUNK
        pltpu.sync_copy(src.at[pl.ds(base, CHUNK)], sidx)
        pltpu.sync_copy(dst.at[pl.ds(base, CHUNK)], didx)
        pltpu.async_copy(table.at[sidx], rows, sem).wait()
        pltpu.sync_copy(rows, acc.at[didx], add=True)
        return 0
      lax.fori_loop(0, N_CHUNKS, step, 0)

    @pl.when(c == 0)
    def _():
      edge_loop(xa)

    @pl.when(c == 1)
    def _():
      edge_loop(xb)

    plsc.subcore_barrier()

    @pl.when(c == 0)
    def _():
      pltpu.sync_copy(acc.at[sl], suma.at[sl])

    @pl.when(c == 1)
    def _():
      pltpu.sync_copy(acc.at[sl], sumb.at[sl])

  return pl.kernel(body, out_type=out_type, mesh=_mesh(),
                   scratch_types=scratch)


@functools.cache
def _build_sc_deg():
  f32 = jnp.float32
  out_type = [
      jax.ShapeDtypeStruct((N_PAD, 128), f32),
      jax.ShapeDtypeStruct((N_PAD, 128), f32),
  ]
  scratch = [
      pltpu.VMEM_SHARED((N_PAD, 128), f32),
      pltpu.VMEM((CHUNK,), jnp.int32),
      pltpu.VMEM((CHUNK, 128), f32),
  ]

  def body(dst, zeros, ones, deg0, deg1, acc, didx, ones_v):
    c = lax.axis_index("c")
    s = lax.axis_index("s")
    w = c * NSUB + s
    sl = pl.ds(s * ROWS_PER_TEC, ROWS_PER_TEC)
    pltpu.sync_copy(zeros.at[sl], acc.at[sl])
    pltpu.sync_copy(ones, ones_v)
    plsc.subcore_barrier()

    def step(i, _):
      base = w * DEG_EDGES_PER_TEC + i * CHUNK
      pltpu.sync_copy(dst.at[pl.ds(base, CHUNK)], didx)
      pltpu.sync_copy(ones_v, acc.at[didx], add=True)
      return 0
    lax.fori_loop(0, DEG_CHUNKS, step, 0)

    plsc.subcore_barrier()

    @pl.when(c == 0)
    def _():
      pltpu.sync_copy(acc.at[sl], deg0.at[sl])

    @pl.when(c == 1)
    def _():
      pltpu.sync_copy(acc.at[sl], deg1.at[sl])

  return pl.kernel(body, out_type=out_type, mesh=_mesh(),
                   scratch_types=scratch)


def _build_tc_layer(out_width: int, relu: bool, split_out: bool):
  R = 1000

  def body(suma, sumb, dega, degb, xa, xb, wl, wr, b, *outs):
    s = jnp.concatenate([suma[...], sumb[...]], axis=1)
    x = jnp.concatenate([xa[...], xb[...]], axis=1)
    rdeg = 1.0 / jnp.maximum(dega[...] + degb[...], 1.0)
    z = jnp.dot(s * rdeg, wl[...], preferred_element_type=jnp.float32,
                precision=lax.Precision.HIGHEST)
    z = z + jnp.dot(x, wr[...], preferred_element_type=jnp.float32,
                    precision=lax.Precision.HIGHEST)
    z = z + b[...]
    if relu:
      z = jnp.maximum(z, 0.0)
    if split_out:
      outs[0][...] = z[:, :128]
      outs[1][...] = z[:, 128:]
    else:
      outs[0][...] = z

  if split_out:
    out_shape = [jax.ShapeDtypeStruct((N_NODES, 128), jnp.float32),
                 jax.ShapeDtypeStruct((N_NODES, 128), jnp.float32)]
    out_specs = [pl.BlockSpec((R, 128), lambda i: (i, 0)),
                 pl.BlockSpec((R, 128), lambda i: (i, 0))]
  else:
    out_shape = [jax.ShapeDtypeStruct((N_NODES, out_width), jnp.float32)]
    out_specs = [pl.BlockSpec((R, out_width), lambda i: (i, 0))]

  grid = (N_NODES // R,)
  in_specs = [
      pl.BlockSpec((R, 128), lambda i: (i, 0)),
      pl.BlockSpec((R, 128), lambda i: (i, 0)),
      pl.BlockSpec((R, 1), lambda i: (i, 0)),
      pl.BlockSpec((R, 1), lambda i: (i, 0)),
      pl.BlockSpec((R, 128), lambda i: (i, 0)),
      pl.BlockSpec((R, 128), lambda i: (i, 0)),
      pl.BlockSpec((256, out_width), lambda i: (0, 0)),
      pl.BlockSpec((256, out_width), lambda i: (0, 0)),
      pl.BlockSpec((1, out_width), lambda i: (0, 0)),
  ]
  return pl.pallas_call(body, grid=grid, in_specs=in_specs,
                        out_specs=out_specs, out_shape=out_shape)


_tc_mid = _build_tc_layer(256, relu=True, split_out=True)
_tc_final = _build_tc_layer(128, relu=False, split_out=False)


@jax.jit
def kernel(x0, x1, edge_index, Wl0, Wr0, b0, Wl1, Wr1, b1,
           Wl2, Wr2, b2, Wl3, Wr3, b3):
  f32 = jnp.float32
  pad_e = E_PAD - N_EDGES
  src = jnp.concatenate([edge_index[0], jnp.zeros((pad_e,), jnp.int32)])
  dst = jnp.concatenate([edge_index[1],
                         jnp.full((pad_e,), N_NODES, jnp.int32)])
  zeros = jnp.zeros((N_PAD, 128), f32)
  ones = jnp.ones((CHUNK, 128), f32)

  z128 = jnp.zeros((128, 128), f32)
  wl01 = jnp.block([[Wl0, z128], [z128, Wl1]])
  wr01 = jnp.block([[Wr0, z128], [z128, Wr1]])
  b01 = jnp.concatenate([b0, b1])[None, :]

  deg0, deg1 = _build_sc_deg()(dst, zeros, ones)
  dega = deg0[:N_NODES, :1]
  degb = deg1[:N_NODES, :1]

  suma, sumb = _build_sc_agg()(x0, x1, src, dst, zeros)
  ha, hb = _tc_mid(suma[:N_NODES], sumb[:N_NODES], dega, degb, x0, x1,
                   wl01, wr01, b01)
  suma, sumb = _build_sc_agg()(ha, hb, src, dst, zeros)
  xa, xb = _tc_mid(suma[:N_NODES], sumb[:N_NODES], dega, degb, ha, hb,
                   Wl2, Wr2, b2[None, :])
  suma, sumb = _build_sc_agg()(xa, xb, src, dst, zeros)
  (out,) = _tc_final(suma[:N_NODES], sumb[:N_NODES], dega, degb, xa, xb,
                     Wl3, Wr3, b3[None, :])
  return out
